# baseline (speedup 1.0000x reference)
import functools

import jax
import jax.numpy as jnp
from jax import lax
from jax.experimental import pallas as pl
from jax.experimental.pallas import tpu as pltpu
from jax.experimental.pallas import tpu_sc as plsc

N_ACTIONS = 18
EPS = 0.99
LO = EPS / N_ACTIONS
HI = 1.0 - EPS + EPS / N_ACTIONS

L = 16
NC, NS = 2, 16
NW = NC * NS
CW = 1024
NCHUNK = 32
CAP = 64
MYCAP = 1024
NDUMP = 16


def kernel(obs, q_table):
    B = obs.shape[0]
    A = q_table.shape[1]
    V = q_table.shape[0]
    b_per_w = B // NW
    vpad = ((V + 127) // 128) * 128

    qT = q_table.T

    mesh = plsc.VectorSubcoreMesh(core_axis_name="c", subcore_axis_name="s")

    @functools.partial(
        pl.kernel,
        out_type=jax.ShapeDtypeStruct((B + NDUMP,), jnp.int32),
        mesh=mesh,
        scratch_types=[
            pltpu.VMEM((B // 128, 128), jnp.int32),
            pltpu.VMEM((MYCAP,), jnp.int32),
            pltpu.VMEM((MYCAP,), jnp.int32),
            pltpu.VMEM((MYCAP,), jnp.int32),
            pltpu.VMEM((NCHUNK,), jnp.int32),
            pltpu.VMEM((NCHUNK, CAP), jnp.int32),
            pltpu.VMEM((NCHUNK, CAP), jnp.int32),
            pltpu.VMEM((2 * (CW // 128) * A, 128), jnp.float32),
            pltpu.SemaphoreType.DMA,
            pltpu.SemaphoreType.DMA,
            pltpu.SemaphoreType.DMA,
        ],
        compiler_params=pltpu.CompilerParams(
            needs_layout_passes=False, use_tc_tiling_on_sc=True
        ),
    )
    def greedy_k(obs_hbm, qT_hbm, gout_hbm, obs_v, my_i, my_p, my_g,
                 ck_n, bk_i, bk_x, chunk_v, sem0, sem1, sem2):
        wid = lax.axis_index("s") * NC + lax.axis_index("c")
        lane = lax.iota(jnp.int32, L)
        zero = jnp.zeros((L,), jnp.int32)

        for r in range(B // 128):
            pltpu.sync_copy(obs_hbm.at[pl.ds(r * 128, 128)], obs_v.at[r])

        def obs_group(g):
            row = g // 8
            col0 = lax.rem(g, 8) * L
            return plsc.load_gather(obs_v, [zero + row, lane + col0])

        def p1(g, cnt):
            iv = obs_group(g)
            mine = (iv // (CW * NCHUNK)) == wid
            pos = cnt + plsc.cumsum(jnp.where(mine, 1, 0)) - 1
            plsc.store_scatter(my_i, [pos], iv, mask=mine)
            plsc.store_scatter(my_p, [pos], lane + g * L, mask=mine)
            return cnt + jnp.sum(jnp.where(mine, 1, 0))

        mycnt = lax.fori_loop(0, B // L, p1, zero)
        ngrp = (mycnt[0] + (L - 1)) // L

        def p2(c, carry):
            cv = zero + c

            def p2g(r, cnt):
                idx = lane + r * L
                valid = idx < mycnt
                iv = plsc.load_gather(my_i, [idx], mask=valid)
                sel = valid & ((iv // CW) == (wid * NCHUNK + c))
                pos = cnt + plsc.cumsum(jnp.where(sel, 1, 0)) - 1
                plsc.store_scatter(bk_i, [cv, pos], iv, mask=sel)
                plsc.store_scatter(bk_x, [cv, pos], idx, mask=sel)
                return cnt + jnp.sum(jnp.where(sel, 1, 0))

            cks = lax.fori_loop(0, ngrp, p2g, zero)
            plsc.store_scatter(ck_n, [cv], cks, mask=lane == 0)
            return carry

        lax.fori_loop(0, NCHUNK, p2, 0)

        def lane0_of(c):
            lane0 = wid * (NCHUNK * CW) + c * CW
            return jnp.minimum(lane0, vpad - CW)

        NP = CW // 128

        def fire(c, buf, sem):
            lane0 = pl.multiple_of(lane0_of(c), 128)
            for j in range(NP):
                pltpu.async_copy(
                    qT_hbm.at[:, pl.ds(lane0 + j * 128, 128)],
                    chunk_v.at[pl.ds((buf * NP + j) * A, A)],
                    sem,
                )

        def drain(buf, sem):
            pltpu.make_async_copy(
                qT_hbm.at[:, pl.ds(0, 128)],
                chunk_v.at[pl.ds(buf * NP * A, NP * A)],
                sem,
            ).wait()

        def compute(c, buf):
            lane0 = lane0_of(c)
            cv = zero + c
            cnt = plsc.load_gather(ck_n, [cv])
            cgrp = (cnt[0] + (L - 1)) // L

            def cg(r, carry):
                slot = lane + r * L
                valid = slot < cnt
                iv = plsc.load_gather(bk_i, [cv, slot], mask=valid)
                xv = plsc.load_gather(bk_x, [cv, slot], mask=valid)
                jv = jnp.where(valid, iv // 128 - lane0 // 128, 0)
                incol = lax.rem(iv, 128)
                row0 = (jv + buf * NP) * A
                best = plsc.load_gather(chunk_v, [row0, incol], mask=valid)
                besta = zero
                for a in range(1, A):
                    ca = jnp.full((L,), a, jnp.int32)
                    va = plsc.load_gather(
                        chunk_v, [row0 + ca, incol], mask=valid)
                    m = va > best
                    best = jnp.where(m, va, best)
                    besta = jnp.where(m, ca, besta)
                plsc.store_scatter(my_g, [xv], besta, mask=valid)
                return carry

            lax.fori_loop(0, cgrp, cg, 0)

        fire(0, 0, sem0)

        def pair_body(p, carry):
            c0 = p * 2
            fire(c0 + 1, 1, sem1)
            drain(0, sem0)
            compute(c0, 0)

            @pl.when(c0 + 2 < NCHUNK)
            def _():
                fire(c0 + 2, 0, sem0)

            drain(1, sem1)
            compute(c0 + 1, 1)
            return carry

        lax.fori_loop(0, NCHUNK // 2, pair_body, 0)

        def p3(r, carry):
            idx = lane + r * L
            valid = idx < mycnt
            pos = plsc.load_gather(my_p, [idx], mask=valid)
            pos = jnp.where(valid, pos, B + lax.rem(idx, NDUMP))
            plsc.store_scatter(my_p, [idx], pos)
            return carry

        lax.fori_loop(0, MYCAP // L, p3, 0)
        pltpu.async_copy(my_g, gout_hbm.at[my_p], sem2).wait()

    greedy = greedy_k(obs, qT)

    @functools.partial(
        pl.kernel,
        out_type=jax.ShapeDtypeStruct((B, A), jnp.float32),
        mesh=mesh,
        scratch_types=[
            pltpu.VMEM((b_per_w // 128, 128), jnp.int32),
            pltpu.VMEM((b_per_w, A), jnp.float32),
            pltpu.SemaphoreType.DMA,
        ],
        compiler_params=pltpu.CompilerParams(
            needs_layout_passes=False, use_tc_tiling_on_sc=False
        ),
    )
    def probs_k(g_hbm, out_hbm, g_v, out_v, sem):
        wid = lax.axis_index("s") * NC + lax.axis_index("c")
        base = wid * b_per_w
        for r in range(b_per_w // 128):
            pltpu.sync_copy(
                g_hbm.at[pl.ds(base + r * 128, 128)], g_v.at[r]
            )
        lo = jnp.full((L,), LO, jnp.float32)
        hi = jnp.full((L,), HI, jnp.float32)
        lane = lax.iota(jnp.int32, L)
        zero = jnp.zeros((L,), jnp.int32)

        def grp(g, carry):
            row = g // 8
            col0 = lax.rem(g, 8) * L
            besta = plsc.load_gather(g_v, [zero + row, lane + col0])
            rows = lane + g * L
            for a in range(A):
                ca = jnp.full((L,), a, jnp.int32)
                vals = jnp.where(besta == ca, hi, lo)
                plsc.store_scatter(out_v, [rows, ca], vals)
            return carry

        lax.fori_loop(0, b_per_w // L, grp, 0)
        pltpu.sync_copy(out_v, out_hbm.at[pl.ds(base, b_per_w)])

    return probs_k(greedy[:B])


# speedup vs baseline: 27.6182x; 27.6182x over previous
import functools

import jax
import jax.numpy as jnp
from jax import lax
from jax.experimental import pallas as pl
from jax.experimental.pallas import tpu as pltpu
from jax.experimental.pallas import tpu_sc as plsc

N_ACTIONS = 18
EPS = 0.99
LO = EPS / N_ACTIONS
HI = 1.0 - EPS + EPS / N_ACTIONS

L = 16
NC, NS = 2, 16
NW = NC * NS
NSLOT = 16


def kernel(obs, q_table):
    B = obs.shape[0]
    A = q_table.shape[1]
    b_per_w = B // NW
    n_groups = b_per_w // L
    n_pairs = n_groups // 2

    qT = q_table.T

    mesh = plsc.VectorSubcoreMesh(core_axis_name="c", subcore_axis_name="s")

    @functools.partial(
        pl.kernel,
        out_type=jax.ShapeDtypeStruct((A, B), jnp.float32),
        mesh=mesh,
        scratch_types=[
            pltpu.VMEM((b_per_w // L, L), jnp.int32),
            pltpu.VMEM((2 * NSLOT * A, 128), jnp.float32),
            pltpu.VMEM((A, b_per_w), jnp.float32),
            pltpu.SemaphoreType.DMA,
            pltpu.SemaphoreType.DMA,
        ],
        compiler_params=pltpu.CompilerParams(
            needs_layout_passes=False, use_tc_tiling_on_sc=True
        ),
    )
    def qpolicy(obs_hbm, qT_hbm, outT_hbm, obs_v, slots_v, out_v,
                sem0, sem1):
        wid = lax.axis_index("s") * NC + lax.axis_index("c")
        base = pl.multiple_of(wid * b_per_w, 128)

        for r in range(b_per_w // L):
            pltpu.sync_copy(
                obs_hbm.at[pl.ds(base + r * L, L)], obs_v.at[r]
            )

        def fire(g, buf, sem):
            iv = obs_v[g, :]
            for k in range(NSLOT):
                i = iv[k]
                tile0 = pl.multiple_of((i // 128) * 128, 128)
                pltpu.async_copy(
                    qT_hbm.at[:, pl.ds(tile0, 128)],
                    slots_v.at[pl.ds((buf * NSLOT + k) * A, A)],
                    sem,
                )

        def drain(buf, sem):
            pltpu.make_async_copy(
                qT_hbm.at[:, pl.ds(0, 128)],
                slots_v.at[pl.ds(buf * NSLOT * A, NSLOT * A)],
                sem,
            ).wait()

        lo = jnp.full((L,), LO, jnp.float32)
        hi = jnp.full((L,), HI, jnp.float32)
        lane = lax.iota(jnp.int32, L)
        zero = jnp.zeros((L,), jnp.int32)

        def compute(g, buf):
            iv = obs_v[g, :]
            incol = lax.rem(iv, 128)
            rowv = (lane + buf * NSLOT) * A
            best = plsc.load_gather(slots_v, [rowv, incol])
            besta = zero
            for a in range(1, A):
                ca = jnp.full((L,), a, jnp.int32)
                va = plsc.load_gather(slots_v, [rowv + ca, incol])
                m = va > best
                best = jnp.where(m, va, best)
                besta = jnp.where(m, ca, besta)
            cols = lane + g * L
            for a in range(A):
                ca = jnp.full((L,), a, jnp.int32)
                vals = jnp.where(besta == ca, hi, lo)
                plsc.store_scatter(out_v, [ca, cols], vals)

        fire(0, 0, sem0)

        def pair_body(p, carry):
            g0 = p * 2
            fire(g0 + 1, 1, sem1)
            drain(0, sem0)
            compute(g0, 0)

            @pl.when(g0 + 2 < n_groups)
            def _():
                fire(g0 + 2, 0, sem0)

            drain(1, sem1)
            compute(g0 + 1, 1)
            return carry

        lax.fori_loop(0, n_pairs, pair_body, 0)
        pltpu.sync_copy(out_v, outT_hbm.at[:, pl.ds(base, b_per_w)])

    return qpolicy(obs, qT).T


# speedup vs baseline: 32.3233x; 1.1704x over previous
import functools

import jax
import jax.numpy as jnp
from jax import lax
from jax.experimental import pallas as pl
from jax.experimental.pallas import tpu as pltpu
from jax.experimental.pallas import tpu_sc as plsc

N_ACTIONS = 18
EPS = 0.99
LO = EPS / N_ACTIONS
HI = 1.0 - EPS + EPS / N_ACTIONS

L = 16
NC, NS = 2, 16
NW = NC * NS
NSLOT = 16


def kernel(obs, q_table):
    B = obs.shape[0]
    A = q_table.shape[1]
    b_per_w = B // NW
    n_groups = b_per_w // L
    n_pairs = n_groups // 2

    qT = q_table.T

    mesh = plsc.VectorSubcoreMesh(core_axis_name="c", subcore_axis_name="s")

    @functools.partial(
        pl.kernel,
        out_type=jax.ShapeDtypeStruct((A, B), jnp.float32),
        mesh=mesh,
        scratch_types=[
            pltpu.VMEM((b_per_w // 128, 128), jnp.int32),
            pltpu.VMEM((2 * NSLOT * A, 128), jnp.float32),
            pltpu.VMEM((A, b_per_w), jnp.float32),
            pltpu.SemaphoreType.DMA,
            pltpu.SemaphoreType.DMA,
        ],
        compiler_params=pltpu.CompilerParams(
            needs_layout_passes=False, use_tc_tiling_on_sc=True
        ),
    )
    def qpolicy(obs_hbm, qT_hbm, outT_hbm, obs_v, slots_v, out_v,
                sem0, sem1):
        wid = lax.axis_index("s") * NC + lax.axis_index("c")
        base = pl.multiple_of(wid * b_per_w, 128)

        obs_copies = [
            pltpu.async_copy(
                obs_hbm.at[pl.ds(base + r * 128, 128)], obs_v.at[r], sem0
            )
            for r in range(b_per_w // 128)
        ]
        for cp in obs_copies:
            cp.wait()

        lane = lax.iota(jnp.int32, L)
        zero = jnp.zeros((L,), jnp.int32)

        def obs_group(g):
            return plsc.load_gather(
                obs_v, [zero + g // 8, lane + lax.rem(g, 8) * L]
            )

        def fire(g, buf, sem):
            iv = obs_group(g)
            for k in range(NSLOT):
                i = iv[k]
                tile0 = pl.multiple_of((i // 128) * 128, 128)
                pltpu.async_copy(
                    qT_hbm.at[:, pl.ds(tile0, 128)],
                    slots_v.at[pl.ds((buf * NSLOT + k) * A, A)],
                    sem,
                )

        def drain(buf, sem):
            pltpu.make_async_copy(
                qT_hbm.at[:, pl.ds(0, 128)],
                slots_v.at[pl.ds(buf * NSLOT * A, NSLOT * A)],
                sem,
            ).wait()

        lo = jnp.full((L,), LO, jnp.float32)
        hi = jnp.full((L,), HI, jnp.float32)

        def compute(g, buf):
            iv = obs_group(g)
            incol = lax.rem(iv, 128)
            rowv = (lane + buf * NSLOT) * A
            best = plsc.load_gather(slots_v, [rowv, incol])
            besta = zero
            for a in range(1, A):
                ca = jnp.full((L,), a, jnp.int32)
                va = plsc.load_gather(slots_v, [rowv + ca, incol])
                m = va > best
                best = jnp.where(m, va, best)
                besta = jnp.where(m, ca, besta)
            cols = lane + g * L
            for a in range(A):
                ca = jnp.full((L,), a, jnp.int32)
                vals = jnp.where(besta == ca, hi, lo)
                plsc.store_scatter(out_v, [ca, cols], vals)

        fire(0, 0, sem0)

        def pair_body(p, carry):
            g0 = p * 2
            fire(g0 + 1, 1, sem1)
            drain(0, sem0)
            compute(g0, 0)

            @pl.when(g0 + 2 < n_groups)
            def _():
                fire(g0 + 2, 0, sem0)

            drain(1, sem1)
            compute(g0 + 1, 1)
            return carry

        lax.fori_loop(0, n_pairs, pair_body, 0)
        pltpu.sync_copy(out_v, outT_hbm.at[:, pl.ds(base, b_per_w)])

    return qpolicy(obs, qT).T
